# baseline (speedup 1.0000x reference)
import functools

import jax
import jax.numpy as jnp
from jax import lax
from jax.experimental import pallas as pl
from jax.experimental.pallas import tpu as pltpu
from jax.experimental.pallas import tpu_sc as plsc

B, T, D = 4, 8192, 768
PAD = 100000
NC, NS = 2, 16
NW = NC * NS
PW = T // NW
C = 32
NCH = PW // C
KV = D // 16
NIT = NCH * B
NB = NIT // 2

_DN = lax.GatherDimensionNumbers(
    offset_dims=(), collapsed_slice_dims=(0,), start_index_map=(0,))

_mesh = plsc.VectorSubcoreMesh(core_axis_name="c", subcore_axis_name="s")


@functools.partial(
    pl.kernel,
    out_type=jax.ShapeDtypeStruct((B * T, D), jnp.float32),
    mesh=_mesh,
    scratch_types=[
        pltpu.VMEM((C,), jnp.int32),
        pltpu.VMEM((2, C), jnp.int32),
        pltpu.VMEM((2, C), jnp.float32),
        pltpu.VMEM((C, D), jnp.float32),
        pltpu.VMEM((2, C, D), jnp.float32),
        pltpu.SemaphoreType.DMA((2,)),
        pltpu.SemaphoreType.DMA((2,)),
        pltpu.SemaphoreType.DMA,
    ],
)
def _emb_lookup(x_hbm, emb_hbm, pos_hbm, out_hbm,
                idxraw, idxs2, mask2, pbuf, ebuf, gsem, osem, psem):
    wid = lax.axis_index("s") * NC + lax.axis_index("c")
    pos_base = wid * PW

    def flat0_of(it):
        return (it % B) * T + pos_base + (it // B) * C

    def prep(it, slot):
        pltpu.sync_copy(x_hbm.at[pl.ds(flat0_of(it), C)], idxraw)
        padv = jnp.zeros((16,), jnp.int32)
        for k in range(C // 16):
            sl = pl.ds(k * 16, 16)
            v = idxraw[sl]
            ispad = v == PAD
            idxs2[slot, sl] = jnp.where(ispad, 0, v)
            mask2[slot, sl] = jnp.where(ispad, 1.0, 0.0)
            padv = padv | jnp.where(ispad, 1, 0)
        lanes = lax.iota(jnp.int32, 16)
        for sh in (8, 4, 2, 1):
            perm = ((lanes + sh) & 15)[:, None]
            padv = padv | lax.gather(
                padv, perm, _DN, (1,),
                mode=lax.GatherScatterMode.PROMISE_IN_BOUNDS)
        return padv[0]

    def start_gather(slot):
        pltpu.async_copy(emb_hbm.at[idxs2.at[slot]], ebuf.at[slot],
                         gsem.at[slot])

    def wait_gather(slot):
        pltpu.make_async_copy(emb_hbm.at[idxs2.at[slot]], ebuf.at[slot],
                              gsem.at[slot]).wait()

    def start_pos(pc):
        pltpu.async_copy(pos_hbm.at[pl.ds(pos_base + pc * C, C)], pbuf, psem)

    def wait_pos(pc):
        pltpu.make_async_copy(pos_hbm.at[pl.ds(pos_base + pc * C, C)],
                              pbuf, psem).wait()

    def start_out(it, slot):
        pltpu.async_copy(ebuf.at[slot], out_hbm.at[pl.ds(flat0_of(it), C)],
                         osem.at[slot])

    def wait_out(it, slot):
        pltpu.make_async_copy(ebuf.at[slot],
                              out_hbm.at[pl.ds(flat0_of(it), C)],
                              osem.at[slot]).wait()

    def compute(slot, anypad):
        def crow(r, c2):
            for k in range(KV):
                sl = pl.ds(k * 16, 16)
                plsc.addupdate(ebuf.at[slot, r, sl], pbuf[r, sl])
            return c2

        lax.fori_loop(0, C, crow, 0)

        @pl.when(anypad != 0)
        def _():
            def rrow(r, c2):
                g16 = pl.multiple_of((r // 16) * 16, 16)
                mv = mask2[slot, pl.ds(g16, 16)]
                m = lax.gather(
                    mv, jnp.full((16, 1), r % 16, jnp.int32), _DN, (1,),
                    mode=lax.GatherScatterMode.PROMISE_IN_BOUNDS)
                km = 1.0 - m

                def rk(k, c3):
                    sl = pl.ds(pl.multiple_of(k * 16, 16), 16)
                    ebuf[slot, r, sl] = (km * ebuf[slot, r, sl]
                                         + m * pbuf[r, sl])
                    return c3

                lax.fori_loop(0, KV, rk, 0)
                return c2

            lax.fori_loop(0, C, rrow, 0)

    start_pos(0)
    apad0 = prep(0, 0)
    start_gather(0)

    def body(i, anypad_e):
        e = 2 * i
        o = e + 1
        pc = i // 2

        @pl.when(i > 0)
        def _():
            wait_out(o - 2, 1)

        anypad_o = prep(o, 1)
        start_gather(1)

        @pl.when(i % 2 == 0)
        def _():
            wait_pos(pc)

        wait_gather(0)
        compute(0, anypad_e)
        start_out(e, 0)

        wait_gather(1)
        compute(1, anypad_o)
        start_out(o, 1)

        @pl.when((i % 2 == 1) & (pc + 1 < NCH))
        def _():
            start_pos(pc + 1)

        wait_out(e, 0)
        anypad_e2 = prep(jnp.minimum(e + 2, NIT - 1), 0)

        @pl.when(i < NB - 1)
        def _():
            start_gather(0)

        return anypad_e2

    lax.fori_loop(0, NB, body, apad0)

    wait_out(NIT - 1, 1)


def kernel(x, emb_table, pos_table):
    out = _emb_lookup(x.reshape(-1).astype(jnp.int32), emb_table, pos_table)
    return out.reshape(B, T, D)
